# baseline (speedup 1.0000x reference)
import functools

import jax
import jax.numpy as jnp
from jax import lax
from jax.experimental import pallas as pl
from jax.experimental.pallas import tpu as pltpu
from jax.experimental.pallas import tpu_sc as plsc

NC = 2
NS = 16
L = 16
NW = NC * NS
CHUNK = 80


def _sc_mesh():
    return plsc.VectorSubcoreMesh(core_axis_name="c", subcore_axis_name="s")


_SC_PARAMS = pltpu.CompilerParams(
    needs_layout_passes=False, use_tc_tiling_on_sc=False)


def _deg_partials(dst_i32, n_nodes):
    e_pad = dst_i32.shape[0]
    epw = e_pad // NW
    n_h = ((n_nodes + 1 + L - 1) // L) * L

    @functools.partial(
        pl.kernel,
        out_type=jax.ShapeDtypeStruct((NW, n_nodes), jnp.float32),
        mesh=_sc_mesh(),
        compiler_params=_SC_PARAMS,
        scratch_types=[
            pltpu.VMEM((epw,), jnp.int32),
            pltpu.VMEM((n_h,), jnp.float32),
        ],
    )
    def deg_kernel(dst_hbm, out_hbm, dst_v, histo_v):
        wid = lax.axis_index("s") * NC + lax.axis_index("c")
        pltpu.sync_copy(dst_hbm.at[pl.ds(wid * epw, epw)], dst_v)

        zero = jnp.zeros((L,), jnp.float32)

        def zbody(i, _):
            histo_v[pl.ds(i * L, L)] = zero
            return 0

        lax.fori_loop(0, n_h // L, zbody, 0)

        ones = jnp.ones((L,), jnp.float32)

        def body(i, _):
            idx = dst_v[pl.ds(i * L, L)]
            plsc.addupdate_scatter(histo_v, [idx], ones)
            return 0

        lax.fori_loop(0, epw // L, body, 0)
        pltpu.sync_copy(histo_v.at[pl.ds(0, n_nodes)], out_hbm.at[wid])

    return deg_kernel(dst_i32)


def _matmul_scale(x, W, degp, bn):
    n, in_ch = x.shape
    out_ch = W.shape[0]
    grid = n // bn

    def body(x_ref, w_ref, degp_ref, h2_ref):
        deg = jnp.sum(degp_ref[...], axis=1) + 1.0
        dis = lax.rsqrt(deg)
        h = lax.dot_general(
            x_ref[...], w_ref[...], (((1,), (1,)), ((), ())),
            preferred_element_type=jnp.float32)
        h2_ref[...] = h * dis[:, None]

    return pl.pallas_call(
        body,
        grid=(grid,),
        in_specs=[
            pl.BlockSpec((bn, in_ch), lambda i: (i, 0)),
            pl.BlockSpec((out_ch, in_ch), lambda i: (0, 0)),
            pl.BlockSpec((bn, NW), lambda i: (i, 0)),
        ],
        out_specs=pl.BlockSpec((bn, out_ch), lambda i: (i, 0)),
        out_shape=jax.ShapeDtypeStruct((n + 8, out_ch), jnp.float32),
    )(x, W, degp)


def _edge_aggregate(src_i32, dst_i32, h2_pad, n_nodes):
    e_pad = src_i32.shape[0]
    epw = e_pad // NW
    n_chunks = epw // CHUNK
    out_ch = h2_pad.shape[1]
    rows_per_tile = n_nodes // NS
    zb = 1
    for d in range(1, 33):
        if rows_per_tile % d == 0:
            zb = d

    @functools.partial(
        pl.kernel,
        out_type=jax.ShapeDtypeStruct((NC, n_nodes, out_ch), jnp.float32),
        mesh=_sc_mesh(),
        compiler_params=_SC_PARAMS,
        scratch_types=[
            pltpu.VMEM((CHUNK,), jnp.int32),
            pltpu.VMEM((CHUNK,), jnp.int32),
            pltpu.VMEM((CHUNK,), jnp.int32),
            pltpu.VMEM((CHUNK,), jnp.int32),
            pltpu.VMEM((CHUNK, out_ch), jnp.float32),
            pltpu.VMEM((CHUNK, out_ch), jnp.float32),
            pltpu.VMEM((zb, out_ch), jnp.float32),
            pltpu.VMEM_SHARED((n_nodes + 8, out_ch), jnp.float32),
            pltpu.VMEM_SHARED((NS, 2, epw), jnp.int32),
            pltpu.SemaphoreType.DMA,
            pltpu.SemaphoreType.DMA,
            pltpu.SemaphoreType.DMA,
            pltpu.SemaphoreType.DMA,
        ],
    )
    def agg_kernel(src_hbm, dst_hbm, h2_hbm, out_hbm,
                   is0, is1, id0, id1, rows0_v, rows1_v, zero_v, acc_sh,
                   idx_sh, sg0, sg1, ss0, ss1):
        c = lax.axis_index("c")
        s = lax.axis_index("s")
        wid = s * NC + c
        base = wid * epw
        rows = (rows0_v, rows1_v)
        isrc = (is0, is1)
        idst = (id0, id1)
        sg = (sg0, sg1)
        ss = (ss0, ss1)

        di0 = pltpu.async_copy(src_hbm.at[pl.ds(base, epw)], idx_sh.at[s, 0], sg0)
        di1 = pltpu.async_copy(dst_hbm.at[pl.ds(base, epw)], idx_sh.at[s, 1], sg1)

        def fetch_idx(k, slot):
            off = k * CHUNK
            pltpu.sync_copy(idx_sh.at[s, 0, pl.ds(off, CHUNK)], isrc[slot])
            pltpu.sync_copy(idx_sh.at[s, 1, pl.ds(off, CHUNK)], idst[slot])

        zero16 = jnp.zeros((L,), jnp.float32)

        def zzero(i, _):
            for j in range(out_ch // L):
                zero_v[i, pl.ds(j * L, L)] = zero16
            return 0

        lax.fori_loop(0, zb, zzero, 0)

        def zfill(i, _):
            pltpu.sync_copy(zero_v, acc_sh.at[pl.ds(s * rows_per_tile + i * zb, zb)])
            return 0

        lax.fori_loop(0, rows_per_tile // zb, zfill, 0)
        di0.wait()
        di1.wait()
        plsc.subcore_barrier()

        padfill = jnp.full((L,), n_nodes, jnp.int32)
        for j in range(CHUNK // L):
            idst[1][pl.ds(j * L, L)] = padfill
        fetch_idx(0, 0)
        pltpu.async_copy(h2_hbm.at[isrc[0]], rows0_v, sg0)
        pltpu.async_copy(rows1_v, acc_sh.at[idst[1]], ss1, add=True)

        def scatter_wait(b):
            pltpu.make_async_copy(rows[b], acc_sh.at[idst[b]], ss[b]).wait()

        def steps(r, _):
            for u in range(2):
                k = r * 2 + u
                b = u
                nb = 1 - u
                scatter_wait(nb)
                fetch_idx(k + 1, nb)
                pltpu.async_copy(h2_hbm.at[isrc[nb]], rows[nb], sg[nb])
                pltpu.make_async_copy(
                    h2_hbm.at[isrc[b]], rows[b], sg[b]).wait()
                pltpu.async_copy(rows[b], acc_sh.at[idst[b]], ss[b], add=True)
            return 0

        lax.fori_loop(0, (n_chunks - 1) // 2, steps, 0)
        scatter_wait(1)
        pltpu.make_async_copy(h2_hbm.at[isrc[0]], rows[0], sg[0]).wait()
        pltpu.async_copy(rows[0], acc_sh.at[idst[0]], ss0, add=True)
        scatter_wait(0)
        plsc.subcore_barrier()

        pltpu.sync_copy(
            acc_sh.at[pl.ds(s * rows_per_tile, rows_per_tile)],
            out_hbm.at[c, pl.ds(s * rows_per_tile, rows_per_tile)])

    return agg_kernel(src_i32, dst_i32, h2_pad)


def _combine(p, h2, degp, b, bn):
    n = degp.shape[0]
    out_ch = h2.shape[1]
    grid = n // bn

    def body(p_ref, h2_ref, degp_ref, b_ref, out_ref):
        deg = jnp.sum(degp_ref[...], axis=1) + 1.0
        dis = lax.rsqrt(deg)
        tot = p_ref[0] + p_ref[1] + h2_ref[...]
        out_ref[...] = tot * dis[:, None] + b_ref[...]

    return pl.pallas_call(
        body,
        grid=(grid,),
        in_specs=[
            pl.BlockSpec((NC, bn, out_ch), lambda i: (0, i, 0)),
            pl.BlockSpec((bn, out_ch), lambda i: (i, 0)),
            pl.BlockSpec((bn, NW), lambda i: (i, 0)),
            pl.BlockSpec((1, out_ch), lambda i: (0, 0)),
        ],
        out_specs=pl.BlockSpec((bn, out_ch), lambda i: (i, 0)),
        out_shape=jax.ShapeDtypeStruct((n, out_ch), jnp.float32),
    )(p, h2, degp, b)


def kernel(x, edge_index, W, b):
    n, _ = x.shape
    out_ch = W.shape[0]
    e = edge_index.shape[1]

    src = edge_index[0].astype(jnp.int32)
    dst = edge_index[1].astype(jnp.int32)

    unit = NW * CHUNK
    e_pad = ((e + unit - 1) // unit) * unit
    if (e_pad // unit) % 2 == 0:
        e_pad += unit
    if e_pad != e:
        pad = jnp.full((e_pad - e,), n, jnp.int32)
        src = jnp.concatenate([src, pad])
        dst = jnp.concatenate([dst, pad])

    degp = _deg_partials(dst, n).T

    bn = 1000 if n % 1000 == 0 else 8
    h2 = _matmul_scale(x, W, degp, bn)

    p = _edge_aggregate(src, dst, h2, n)

    return _combine(p, h2, degp, b.reshape(1, out_ch), bn)


# speedup vs baseline: 1.0252x; 1.0252x over previous
import functools

import jax
import jax.numpy as jnp
from jax import lax
from jax.experimental import pallas as pl
from jax.experimental.pallas import tpu as pltpu
from jax.experimental.pallas import tpu_sc as plsc

NC = 2
NS = 16
L = 16
NW = NC * NS
CHUNK = 80


def _sc_mesh():
    return plsc.VectorSubcoreMesh(core_axis_name="c", subcore_axis_name="s")


_SC_PARAMS = pltpu.CompilerParams(
    needs_layout_passes=False, use_tc_tiling_on_sc=False)


def _deg_partials(dst_i32, n_nodes):
    e_pad = dst_i32.shape[0]
    epw = e_pad // NW
    n_h = ((n_nodes + 1 + L - 1) // L) * L

    @functools.partial(
        pl.kernel,
        out_type=jax.ShapeDtypeStruct((NW, n_nodes), jnp.float32),
        mesh=_sc_mesh(),
        compiler_params=_SC_PARAMS,
        scratch_types=[
            pltpu.VMEM((epw,), jnp.int32),
            pltpu.VMEM((n_h,), jnp.float32),
        ],
    )
    def deg_kernel(dst_hbm, out_hbm, dst_v, histo_v):
        wid = lax.axis_index("s") * NC + lax.axis_index("c")
        pltpu.sync_copy(dst_hbm.at[pl.ds(wid * epw, epw)], dst_v)

        zero = jnp.zeros((L,), jnp.float32)

        def zbody(i, _):
            histo_v[pl.ds(i * L, L)] = zero
            return 0

        lax.fori_loop(0, n_h // L, zbody, 0)

        ones = jnp.ones((L,), jnp.float32)

        def body(i, _):
            idx = dst_v[pl.ds(i * L, L)]
            plsc.addupdate_scatter(histo_v, [idx], ones)
            return 0

        lax.fori_loop(0, epw // L, body, 0)
        pltpu.sync_copy(histo_v.at[pl.ds(0, n_nodes)], out_hbm.at[wid])

    return deg_kernel(dst_i32)


def _matmul_scale(x, W, degp, bn):
    n, in_ch = x.shape
    out_ch = W.shape[0]
    grid = n // bn

    def body(x_ref, w_ref, degp_ref, h2_ref):
        deg = jnp.sum(degp_ref[...], axis=1) + 1.0
        dis = lax.rsqrt(deg)
        h = lax.dot_general(
            x_ref[...], w_ref[...], (((1,), (1,)), ((), ())),
            preferred_element_type=jnp.float32)
        h2_ref[...] = h * dis[:, None]

    return pl.pallas_call(
        body,
        grid=(grid,),
        in_specs=[
            pl.BlockSpec((bn, in_ch), lambda i: (i, 0)),
            pl.BlockSpec((out_ch, in_ch), lambda i: (0, 0)),
            pl.BlockSpec((bn, NW), lambda i: (i, 0)),
        ],
        out_specs=pl.BlockSpec((bn, out_ch), lambda i: (i, 0)),
        out_shape=jax.ShapeDtypeStruct((n + 8, out_ch), jnp.float32),
    )(x, W, degp)


def _edge_aggregate(src_i32, dst_i32, h2_pad, n_nodes):
    e_pad = src_i32.shape[0]
    epw = e_pad // NW
    n_chunks = epw // CHUNK
    out_ch = h2_pad.shape[1]
    rows_per_tile = n_nodes // NS
    zb = 1
    for d in range(1, 33):
        if rows_per_tile % d == 0:
            zb = d

    @functools.partial(
        pl.kernel,
        out_type=jax.ShapeDtypeStruct((NC, n_nodes, out_ch), jnp.float32),
        mesh=_sc_mesh(),
        compiler_params=_SC_PARAMS,
        scratch_types=[
            pltpu.VMEM((CHUNK,), jnp.int32),
            pltpu.VMEM((CHUNK,), jnp.int32),
            pltpu.VMEM((CHUNK,), jnp.int32),
            pltpu.VMEM((CHUNK,), jnp.int32),
            pltpu.VMEM((CHUNK, out_ch), jnp.float32),
            pltpu.VMEM((CHUNK, out_ch), jnp.float32),
            pltpu.VMEM((zb, out_ch), jnp.float32),
            pltpu.VMEM_SHARED((n_nodes + 8, out_ch), jnp.float32),
            pltpu.VMEM_SHARED((NS, 2, epw), jnp.int32),
            pltpu.SemaphoreType.DMA,
            pltpu.SemaphoreType.DMA,
        ],
    )
    def agg_kernel(src_hbm, dst_hbm, h2_hbm, out_hbm,
                   is0, is1, id0, id1, rows0_v, rows1_v, zero_v, acc_sh,
                   idx_sh, sg0, sg1):
        c = lax.axis_index("c")
        s = lax.axis_index("s")
        wid = s * NC + c
        base = wid * epw
        rows = (rows0_v, rows1_v)
        isrc = (is0, is1)
        idst = (id0, id1)
        sg = (sg0, sg1)

        di0 = pltpu.async_copy(src_hbm.at[pl.ds(base, epw)], idx_sh.at[s, 0], sg0)
        di1 = pltpu.async_copy(dst_hbm.at[pl.ds(base, epw)], idx_sh.at[s, 1], sg1)

        def fetch_idx(k, slot):
            off = k * CHUNK
            pltpu.sync_copy(idx_sh.at[s, 0, pl.ds(off, CHUNK)], isrc[slot])
            pltpu.sync_copy(idx_sh.at[s, 1, pl.ds(off, CHUNK)], idst[slot])

        zero16 = jnp.zeros((L,), jnp.float32)

        def zzero(i, _):
            for j in range(out_ch // L):
                zero_v[i, pl.ds(j * L, L)] = zero16
            return 0

        lax.fori_loop(0, zb, zzero, 0)

        def zfill(i, _):
            pltpu.sync_copy(zero_v, acc_sh.at[pl.ds(s * rows_per_tile + i * zb, zb)])
            return 0

        lax.fori_loop(0, rows_per_tile // zb, zfill, 0)
        di0.wait()
        di1.wait()
        plsc.subcore_barrier()

        fetch_idx(0, 0)
        pltpu.async_copy(h2_hbm.at[isrc[0]], rows0_v, sg0)

        def steps(r, _):
            for u in range(2):
                k = r * 2 + u
                b = u
                nb = 1 - u
                fetch_idx(k + 1, nb)
                pltpu.async_copy(h2_hbm.at[isrc[nb]], rows[nb], sg[nb])
                pltpu.make_async_copy(
                    h2_hbm.at[isrc[b]], rows[b], sg[b]).wait()
                pltpu.sync_copy(rows[b], acc_sh.at[idst[b]], add=True)
            return 0

        lax.fori_loop(0, (n_chunks - 1) // 2, steps, 0)
        pltpu.make_async_copy(h2_hbm.at[isrc[0]], rows[0], sg[0]).wait()
        pltpu.sync_copy(rows[0], acc_sh.at[idst[0]], add=True)
        plsc.subcore_barrier()

        pltpu.sync_copy(
            acc_sh.at[pl.ds(s * rows_per_tile, rows_per_tile)],
            out_hbm.at[c, pl.ds(s * rows_per_tile, rows_per_tile)])

    return agg_kernel(src_i32, dst_i32, h2_pad)


def _combine(p, h2, degp, b, bn):
    n = degp.shape[0]
    out_ch = h2.shape[1]
    grid = n // bn

    def body(p_ref, h2_ref, degp_ref, b_ref, out_ref):
        deg = jnp.sum(degp_ref[...], axis=1) + 1.0
        dis = lax.rsqrt(deg)
        tot = p_ref[0] + p_ref[1] + h2_ref[...]
        out_ref[...] = tot * dis[:, None] + b_ref[...]

    return pl.pallas_call(
        body,
        grid=(grid,),
        in_specs=[
            pl.BlockSpec((NC, bn, out_ch), lambda i: (0, i, 0)),
            pl.BlockSpec((bn, out_ch), lambda i: (i, 0)),
            pl.BlockSpec((bn, NW), lambda i: (i, 0)),
            pl.BlockSpec((1, out_ch), lambda i: (0, 0)),
        ],
        out_specs=pl.BlockSpec((bn, out_ch), lambda i: (i, 0)),
        out_shape=jax.ShapeDtypeStruct((n, out_ch), jnp.float32),
    )(p, h2, degp, b)


def kernel(x, edge_index, W, b):
    n, _ = x.shape
    out_ch = W.shape[0]
    e = edge_index.shape[1]

    src = edge_index[0].astype(jnp.int32)
    dst = edge_index[1].astype(jnp.int32)

    unit = NW * CHUNK
    e_pad = ((e + unit - 1) // unit) * unit
    if (e_pad // unit) % 2 == 0:
        e_pad += unit
    if e_pad != e:
        pad = jnp.full((e_pad - e,), n, jnp.int32)
        src = jnp.concatenate([src, pad])
        dst = jnp.concatenate([dst, pad])

    degp = _deg_partials(dst, n).T

    bn = 2000 if n % 2000 == 0 else (1000 if n % 1000 == 0 else 8)
    h2 = _matmul_scale(x, W, degp, bn)

    p = _edge_aggregate(src, dst, h2, n)

    return _combine(p, h2, degp, b.reshape(1, out_ch), bn)
